# baseline (speedup 1.0000x reference)
import functools

import jax
import jax.numpy as jnp
from jax import lax
from jax.experimental import pallas as pl
from jax.experimental.pallas import tpu as pltpu

try:
    from jax.experimental.pallas import tpu_sc as plsc
except ImportError:
    plsc = None

NT = 2304
KT = 2048


def _argmin_body(nk, z_ref, cb_ref, idx_out_ref, bd_ref, bi_ref, esq_ref):
    k = pl.program_id(0)
    n = pl.program_id(1)
    nchunk = KT // 128

    z = z_ref[...]
    cb = cb_ref[...]

    @pl.when(n == 0)
    def _esq():
        esq_ref[...] = jnp.sum(cb * cb, axis=1)

    z_sq = jnp.sum(z * z, axis=1, keepdims=True)
    e_sq = esq_ref[...][None, :]
    prod2 = lax.dot_general(z + z, cb, (((1,), (1,)), ((), ())),
                            preferred_element_type=jnp.float32)
    dist = (z_sq - prod2) + e_sq

    sl = pl.ds(n * NT, NT)

    @pl.when(k == 0)
    def _init():
        bd_ref[sl, :] = jnp.full((NT, 128), jnp.inf, jnp.float32)
        bi_ref[sl, :] = jnp.zeros((NT, 128), jnp.int32)

    bd = bd_ref[sl, :]
    bi = bi_ref[sl, :]
    for c in range(nchunk):
        cand = dist[:, c * 128:(c + 1) * 128]
        chunk_id = k * nchunk + c
        take = cand < bd
        bd = jnp.where(take, cand, bd)
        bi = jnp.where(take, chunk_id, bi)
    bd_ref[sl, :] = bd
    bi_ref[sl, :] = bi

    @pl.when(k == nk - 1)
    def _extract():
        lane = lax.broadcasted_iota(jnp.int32, (NT, 128), 1)
        full_idx = bi * 128 + lane
        m = jnp.min(bd, axis=1)
        masked = jnp.where(bd == m[:, None], full_idx, jnp.int32(1 << 30))
        idx_out_ref[sl] = jnp.min(masked, axis=1)


def _compute_indices(zf, codebook, n_start, n_count):
    d = zf.shape[1]
    n_codes = codebook.shape[0]
    nk = n_codes // KT
    n_out = n_count * NT

    grid_spec = pltpu.PrefetchScalarGridSpec(
        num_scalar_prefetch=0,
        grid=(nk, n_count),
        in_specs=[
            pl.BlockSpec((NT, d), lambda k, n: (n_start + n, 0)),
            pl.BlockSpec((KT, d), lambda k, n: (k, 0)),
        ],
        out_specs=pl.BlockSpec((n_out,), lambda k, n: (0,)),
        scratch_shapes=[
            pltpu.VMEM((n_out, 128), jnp.float32),
            pltpu.VMEM((n_out, 128), jnp.int32),
            pltpu.VMEM((KT,), jnp.float32),
        ],
    )
    return pl.pallas_call(
        functools.partial(_argmin_body, nk),
        grid_spec=grid_spec,
        out_shape=jax.ShapeDtypeStruct((n_out,), jnp.int32),
        compiler_params=pltpu.CompilerParams(
            dimension_semantics=("arbitrary", "arbitrary"),
        ),
    )(zf, codebook)


def _sc_gather(codebook, indices):
    n_tokens = indices.shape[0]
    d = codebook.shape[1]
    info = plsc.get_sparse_core_info()
    nc, ns = info.num_cores, info.num_subcores
    nw = nc * ns
    b_per_w = n_tokens // nw
    mesh = plsc.VectorSubcoreMesh(core_axis_name="c", subcore_axis_name="s")

    @functools.partial(
        pl.kernel,
        mesh=mesh,
        out_type=jax.ShapeDtypeStruct((n_tokens, d), jnp.float32),
        scratch_types=[
            pltpu.VMEM((b_per_w,), jnp.int32),
            pltpu.VMEM((b_per_w, d), jnp.float32),
            pltpu.SemaphoreType.DMA,
        ],
    )
    def gather_kernel(table_hbm, idx_hbm, out_hbm, idx_v, rows_v, sem):
        wid = lax.axis_index("s") * nc + lax.axis_index("c")
        base = wid * b_per_w
        pltpu.sync_copy(idx_hbm.at[pl.ds(base, b_per_w)], idx_v)
        pltpu.async_copy(table_hbm.at[idx_v], rows_v, sem).wait()
        pltpu.sync_copy(rows_v, out_hbm.at[pl.ds(base, b_per_w)])

    return gather_kernel(codebook, indices)


def kernel(z, codebook):
    b, t, d = z.shape
    zf = z.reshape(-1, d)
    nn = zf.shape[0] // NT
    h = nn // 2
    i1 = _compute_indices(zf, codebook, 0, h)
    q1 = _sc_gather(codebook, i1)
    i2 = _compute_indices(zf, codebook, h, nn - h)
    q2 = _sc_gather(codebook, i2)
    indices = jnp.concatenate([i1, i2])
    quantized = jnp.concatenate([q1, q2])
    return quantized.reshape(b, t, d), indices.reshape(b, t)


# speedup vs baseline: 1.0333x; 1.0333x over previous
import functools

import jax
import jax.numpy as jnp
from jax import lax
from jax.experimental import pallas as pl
from jax.experimental.pallas import tpu as pltpu

try:
    from jax.experimental.pallas import tpu_sc as plsc
except ImportError:
    plsc = None

NT = 2304
KT = 2048


def _argmin_body(nk, z_ref, cb_ref, idx_out_ref, bd_ref, bi_ref, esq_ref):
    n = pl.program_id(0)
    k = pl.program_id(1)
    nchunk = KT // 128

    z = z_ref[...]
    cb = cb_ref[...]

    ks = pl.ds(k * KT, KT)

    @pl.when(n == 0)
    def _esq():
        esq_ref[ks] = jnp.sum(cb * cb, axis=1)

    z_sq = jnp.sum(z * z, axis=1, keepdims=True)
    e_sq = esq_ref[ks][None, :]
    prod2 = lax.dot_general(z + z, cb, (((1,), (1,)), ((), ())),
                            preferred_element_type=jnp.float32)
    dist = (z_sq - prod2) + e_sq

    @pl.when(k == 0)
    def _init():
        bd_ref[...] = jnp.full((NT, 128), jnp.inf, jnp.float32)
        bi_ref[...] = jnp.zeros((NT, 128), jnp.int32)

    bd = bd_ref[...]
    bi = bi_ref[...]
    for c in range(nchunk):
        cand = dist[:, c * 128:(c + 1) * 128]
        chunk_id = k * nchunk + c
        take = cand < bd
        bd = jnp.where(take, cand, bd)
        bi = jnp.where(take, chunk_id, bi)
    bd_ref[...] = bd
    bi_ref[...] = bi

    @pl.when(k == nk - 1)
    def _extract():
        lane = lax.broadcasted_iota(jnp.int32, (NT, 128), 1)
        full_idx = bi * 128 + lane
        m = jnp.min(bd, axis=1)
        masked = jnp.where(bd == m[:, None], full_idx, jnp.int32(1 << 30))
        idx_out_ref[0, 0, :] = jnp.min(masked, axis=1)


def _compute_indices(zf, codebook, n_start, n_count):
    d = zf.shape[1]
    n_codes = codebook.shape[0]
    nk = n_codes // KT
    n_out = n_count * NT

    grid_spec = pltpu.PrefetchScalarGridSpec(
        num_scalar_prefetch=0,
        grid=(n_count, nk),
        in_specs=[
            pl.BlockSpec((NT, d), lambda n, k: (n_start + n, 0)),
            pl.BlockSpec((KT, d), lambda n, k: (k, 0)),
        ],
        out_specs=pl.BlockSpec((1, 1, NT), lambda n, k: (n, 0, 0)),
        scratch_shapes=[
            pltpu.VMEM((NT, 128), jnp.float32),
            pltpu.VMEM((NT, 128), jnp.int32),
            pltpu.VMEM((n_codes,), jnp.float32),
        ],
    )
    out = pl.pallas_call(
        functools.partial(_argmin_body, nk),
        grid_spec=grid_spec,
        out_shape=jax.ShapeDtypeStruct((n_count, 1, NT), jnp.int32),
        compiler_params=pltpu.CompilerParams(
            dimension_semantics=("arbitrary", "arbitrary"),
        ),
    )(zf, codebook)
    return out.reshape(n_out)


def _sc_gather(codebook, indices):
    n_tokens = indices.shape[0]
    d = codebook.shape[1]
    info = plsc.get_sparse_core_info()
    nc, ns = info.num_cores, info.num_subcores
    nw = nc * ns
    b_per_w = n_tokens // nw
    mesh = plsc.VectorSubcoreMesh(core_axis_name="c", subcore_axis_name="s")

    @functools.partial(
        pl.kernel,
        mesh=mesh,
        out_type=jax.ShapeDtypeStruct((n_tokens, d), jnp.float32),
        scratch_types=[
            pltpu.VMEM((b_per_w,), jnp.int32),
            pltpu.VMEM((b_per_w, d), jnp.float32),
            pltpu.SemaphoreType.DMA,
        ],
    )
    def gather_kernel(table_hbm, idx_hbm, out_hbm, idx_v, rows_v, sem):
        wid = lax.axis_index("s") * nc + lax.axis_index("c")
        base = wid * b_per_w
        pltpu.sync_copy(idx_hbm.at[pl.ds(base, b_per_w)], idx_v)
        pltpu.async_copy(table_hbm.at[idx_v], rows_v, sem).wait()
        pltpu.sync_copy(rows_v, out_hbm.at[pl.ds(base, b_per_w)])

    return gather_kernel(codebook, indices)


def kernel(z, codebook):
    b, t, d = z.shape
    zf = z.reshape(-1, d)
    nn = zf.shape[0] // NT
    indices = _compute_indices(zf, codebook, 0, nn)
    quantized = _sc_gather(codebook, indices)
    return quantized.reshape(b, t, d), indices.reshape(b, t)


# speedup vs baseline: 1.0799x; 1.0451x over previous
import functools

import jax
import jax.numpy as jnp
from jax import lax
from jax.experimental import pallas as pl
from jax.experimental.pallas import tpu as pltpu

try:
    from jax.experimental.pallas import tpu_sc as plsc
except ImportError:
    plsc = None

NT = 2304
KT = 2048


def _argmin_body(nk, z_ref, cb_ref, idx_out_ref, bd_ref, bi_ref, esq_ref):
    k = pl.program_id(0)
    n = pl.program_id(1)
    nchunk = KT // 128

    z = z_ref[...]
    cb = cb_ref[...]

    @pl.when(n == 0)
    def _esq():
        esq_ref[...] = jnp.sum(cb * cb, axis=1)

    z_sq = jnp.sum(z * z, axis=1, keepdims=True)
    e_sq = esq_ref[...][None, :]
    prod2 = lax.dot_general(z + z, cb, (((1,), (1,)), ((), ())),
                            preferred_element_type=jnp.float32)
    dist = (z_sq - prod2) + e_sq

    sl = pl.ds(n * NT, NT)

    @pl.when(k == 0)
    def _init():
        bd_ref[sl, :] = jnp.full((NT, 128), jnp.inf, jnp.float32)
        bi_ref[sl, :] = jnp.zeros((NT, 128), jnp.int32)

    bd = bd_ref[sl, :]
    bi = bi_ref[sl, :]
    for c in range(nchunk):
        cand = dist[:, c * 128:(c + 1) * 128]
        chunk_id = k * nchunk + c
        take = cand < bd
        bd = jnp.minimum(cand, bd)
        bi = jnp.where(take, chunk_id, bi)
    bd_ref[sl, :] = bd
    bi_ref[sl, :] = bi

    @pl.when(k == nk - 1)
    def _extract():
        lane = lax.broadcasted_iota(jnp.int32, (NT, 128), 1)
        full_idx = bi * 128 + lane
        m = jnp.min(bd, axis=1)
        masked = jnp.where(bd == m[:, None], full_idx, jnp.int32(1 << 30))
        idx_out_ref[sl] = jnp.min(masked, axis=1)


def _compute_indices(zf, codebook, n_start, n_count):
    d = zf.shape[1]
    n_codes = codebook.shape[0]
    nk = n_codes // KT
    n_out = n_count * NT

    grid_spec = pltpu.PrefetchScalarGridSpec(
        num_scalar_prefetch=0,
        grid=(nk, n_count),
        in_specs=[
            pl.BlockSpec((NT, d), lambda k, n: (n_start + n, 0)),
            pl.BlockSpec((KT, d), lambda k, n: (k, 0)),
        ],
        out_specs=pl.BlockSpec((n_out,), lambda k, n: (0,)),
        scratch_shapes=[
            pltpu.VMEM((n_out, 128), jnp.float32),
            pltpu.VMEM((n_out, 128), jnp.int32),
            pltpu.VMEM((KT,), jnp.float32),
        ],
    )
    return pl.pallas_call(
        functools.partial(_argmin_body, nk),
        grid_spec=grid_spec,
        out_shape=jax.ShapeDtypeStruct((n_out,), jnp.int32),
        compiler_params=pltpu.CompilerParams(
            dimension_semantics=("arbitrary", "arbitrary"),
        ),
    )(zf, codebook)


def _sc_gather(codebook, indices):
    n_tokens = indices.shape[0]
    d = codebook.shape[1]
    info = plsc.get_sparse_core_info()
    nc, ns = info.num_cores, info.num_subcores
    nw = nc * ns
    b_per_w = n_tokens // nw
    mesh = plsc.VectorSubcoreMesh(core_axis_name="c", subcore_axis_name="s")

    @functools.partial(
        pl.kernel,
        mesh=mesh,
        out_type=jax.ShapeDtypeStruct((n_tokens, d), jnp.float32),
        scratch_types=[
            pltpu.VMEM((b_per_w,), jnp.int32),
            pltpu.VMEM((b_per_w, d), jnp.float32),
            pltpu.SemaphoreType.DMA,
        ],
    )
    def gather_kernel(table_hbm, idx_hbm, out_hbm, idx_v, rows_v, sem):
        wid = lax.axis_index("s") * nc + lax.axis_index("c")
        base = wid * b_per_w
        pltpu.sync_copy(idx_hbm.at[pl.ds(base, b_per_w)], idx_v)
        pltpu.async_copy(table_hbm.at[idx_v], rows_v, sem).wait()
        pltpu.sync_copy(rows_v, out_hbm.at[pl.ds(base, b_per_w)])

    return gather_kernel(codebook, indices)


def kernel(z, codebook):
    b, t, d = z.shape
    zf = z.reshape(-1, d)
    nn = zf.shape[0] // NT
    indices = _compute_indices(zf, codebook, 0, nn)
    quantized = _sc_gather(codebook, indices)
    return quantized.reshape(b, t, d), indices.reshape(b, t)


# speedup vs baseline: 1.2175x; 1.1275x over previous
import functools

import jax
import jax.numpy as jnp
from jax import lax
from jax.experimental import pallas as pl
from jax.experimental.pallas import tpu as pltpu

try:
    from jax.experimental.pallas import tpu_sc as plsc
except ImportError:
    plsc = None

NT = 2304
KT = 2048


def _argmin_body(nk, z_ref, cb_ref, idx_out_ref, bd_ref, bi_ref, esq_ref):
    k = pl.program_id(0)
    n = pl.program_id(1)
    nchunk = KT // 128

    z = z_ref[...]
    cb = cb_ref[...]

    @pl.when(n == 0)
    def _esq():
        esq_ref[...] = jnp.sum(cb * cb, axis=1)

    z_sq = jnp.sum(z * z, axis=1, keepdims=True)
    e_sq = esq_ref[...][None, :]
    z2 = z + z
    half = KT // 2

    sl = pl.ds(n * NT, NT)

    @pl.when(k == 0)
    def _init():
        bd_ref[sl, :] = jnp.full((NT, 128), jnp.inf, jnp.float32)
        bi_ref[sl, :] = jnp.zeros((NT, 128), jnp.int32)

    bd = bd_ref[sl, :]
    bi = bi_ref[sl, :]
    for h in range(2):
        lo = h * half
        prod2 = lax.dot_general(z2, cb[lo:lo + half, :],
                                (((1,), (1,)), ((), ())),
                                preferred_element_type=jnp.float32)
        dist = (z_sq - prod2) + e_sq[:, lo:lo + half]
        for c in range(half // 128):
            cand = dist[:, c * 128:(c + 1) * 128]
            chunk_id = k * nchunk + h * (half // 128) + c
            take = cand < bd
            bd = jnp.minimum(cand, bd)
            bi = jnp.where(take, chunk_id, bi)
    bd_ref[sl, :] = bd
    bi_ref[sl, :] = bi

    @pl.when(k == nk - 1)
    def _extract():
        lane = lax.broadcasted_iota(jnp.int32, (NT, 128), 1)
        full_idx = bi * 128 + lane
        m = jnp.min(bd, axis=1)
        masked = jnp.where(bd == m[:, None], full_idx, jnp.int32(1 << 30))
        idx_out_ref[sl] = jnp.min(masked, axis=1)


def _compute_indices(zf, codebook, n_start, n_count):
    d = zf.shape[1]
    n_codes = codebook.shape[0]
    nk = n_codes // KT
    n_out = n_count * NT

    grid_spec = pltpu.PrefetchScalarGridSpec(
        num_scalar_prefetch=0,
        grid=(nk, n_count),
        in_specs=[
            pl.BlockSpec((NT, d), lambda k, n: (n_start + n, 0)),
            pl.BlockSpec((KT, d), lambda k, n: (k, 0)),
        ],
        out_specs=pl.BlockSpec((n_out,), lambda k, n: (0,)),
        scratch_shapes=[
            pltpu.VMEM((n_out, 128), jnp.float32),
            pltpu.VMEM((n_out, 128), jnp.int32),
            pltpu.VMEM((KT,), jnp.float32),
        ],
    )
    return pl.pallas_call(
        functools.partial(_argmin_body, nk),
        grid_spec=grid_spec,
        out_shape=jax.ShapeDtypeStruct((n_out,), jnp.int32),
        compiler_params=pltpu.CompilerParams(
            dimension_semantics=("arbitrary", "arbitrary"),
        ),
    )(zf, codebook)


def _sc_gather(codebook, indices):
    n_tokens = indices.shape[0]
    d = codebook.shape[1]
    info = plsc.get_sparse_core_info()
    nc, ns = info.num_cores, info.num_subcores
    nw = nc * ns
    b_per_w = n_tokens // nw
    mesh = plsc.VectorSubcoreMesh(core_axis_name="c", subcore_axis_name="s")

    @functools.partial(
        pl.kernel,
        mesh=mesh,
        out_type=jax.ShapeDtypeStruct((n_tokens, d), jnp.float32),
        scratch_types=[
            pltpu.VMEM((b_per_w,), jnp.int32),
            pltpu.VMEM((b_per_w, d), jnp.float32),
            pltpu.SemaphoreType.DMA,
        ],
    )
    def gather_kernel(table_hbm, idx_hbm, out_hbm, idx_v, rows_v, sem):
        wid = lax.axis_index("s") * nc + lax.axis_index("c")
        base = wid * b_per_w
        pltpu.sync_copy(idx_hbm.at[pl.ds(base, b_per_w)], idx_v)
        pltpu.async_copy(table_hbm.at[idx_v], rows_v, sem).wait()
        pltpu.sync_copy(rows_v, out_hbm.at[pl.ds(base, b_per_w)])

    return gather_kernel(codebook, indices)


def kernel(z, codebook):
    b, t, d = z.shape
    zf = z.reshape(-1, d)
    nn = zf.shape[0] // NT
    indices = _compute_indices(zf, codebook, 0, nn)
    quantized = _sc_gather(codebook, indices)
    return quantized.reshape(b, t, d), indices.reshape(b, t)


# speedup vs baseline: 1.2539x; 1.0299x over previous
import functools

import jax
import jax.numpy as jnp
from jax import lax
from jax.experimental import pallas as pl
from jax.experimental.pallas import tpu as pltpu

try:
    from jax.experimental.pallas import tpu_sc as plsc
except ImportError:
    plsc = None

NT = 2304
KT = 4096


def _argmin_body(nk, z_ref, cb_ref, idx_out_ref, bd_ref, bi_ref, esq_ref):
    k = pl.program_id(0)
    n = pl.program_id(1)
    nchunk = KT // 128

    z = z_ref[...]
    cb = cb_ref[...]

    @pl.when(n == 0)
    def _esq():
        esq_ref[...] = jnp.sum(cb * cb, axis=1)

    z_sq = jnp.sum(z * z, axis=1, keepdims=True)
    e_sq = esq_ref[...][None, :]
    z2 = z + z
    half = KT // 4

    sl = pl.ds(n * NT, NT)

    @pl.when(k == 0)
    def _init():
        bd_ref[sl, :] = jnp.full((NT, 128), jnp.inf, jnp.float32)
        bi_ref[sl, :] = jnp.zeros((NT, 128), jnp.int32)

    bd = bd_ref[sl, :]
    bi = bi_ref[sl, :]
    for h in range(4):
        lo = h * half
        prod2 = lax.dot_general(z2, cb[lo:lo + half, :],
                                (((1,), (1,)), ((), ())),
                                preferred_element_type=jnp.float32)
        dist = (z_sq - prod2) + e_sq[:, lo:lo + half]
        for c in range(half // 128):
            cand = dist[:, c * 128:(c + 1) * 128]
            chunk_id = k * nchunk + h * (half // 128) + c
            take = cand < bd
            bd = jnp.minimum(cand, bd)
            bi = jnp.where(take, chunk_id, bi)
    bd_ref[sl, :] = bd
    bi_ref[sl, :] = bi

    @pl.when(k == nk - 1)
    def _extract():
        lane = lax.broadcasted_iota(jnp.int32, (NT, 128), 1)
        full_idx = bi * 128 + lane
        m = jnp.min(bd, axis=1)
        masked = jnp.where(bd == m[:, None], full_idx, jnp.int32(1 << 30))
        idx_out_ref[sl] = jnp.min(masked, axis=1)


def _compute_indices(zf, codebook, n_start, n_count):
    d = zf.shape[1]
    n_codes = codebook.shape[0]
    nk = n_codes // KT
    n_out = n_count * NT

    grid_spec = pltpu.PrefetchScalarGridSpec(
        num_scalar_prefetch=0,
        grid=(nk, n_count),
        in_specs=[
            pl.BlockSpec((NT, d), lambda k, n: (n_start + n, 0)),
            pl.BlockSpec((KT, d), lambda k, n: (k, 0)),
        ],
        out_specs=pl.BlockSpec((n_out,), lambda k, n: (0,)),
        scratch_shapes=[
            pltpu.VMEM((n_out, 128), jnp.float32),
            pltpu.VMEM((n_out, 128), jnp.int32),
            pltpu.VMEM((KT,), jnp.float32),
        ],
    )
    return pl.pallas_call(
        functools.partial(_argmin_body, nk),
        grid_spec=grid_spec,
        out_shape=jax.ShapeDtypeStruct((n_out,), jnp.int32),
        compiler_params=pltpu.CompilerParams(
            dimension_semantics=("arbitrary", "arbitrary"),
        ),
    )(zf, codebook)


def _sc_gather(codebook, indices):
    n_tokens = indices.shape[0]
    d = codebook.shape[1]
    info = plsc.get_sparse_core_info()
    nc, ns = info.num_cores, info.num_subcores
    nw = nc * ns
    b_per_w = n_tokens // nw
    mesh = plsc.VectorSubcoreMesh(core_axis_name="c", subcore_axis_name="s")

    @functools.partial(
        pl.kernel,
        mesh=mesh,
        out_type=jax.ShapeDtypeStruct((n_tokens, d), jnp.float32),
        scratch_types=[
            pltpu.VMEM((b_per_w,), jnp.int32),
            pltpu.VMEM((b_per_w, d), jnp.float32),
            pltpu.SemaphoreType.DMA,
        ],
    )
    def gather_kernel(table_hbm, idx_hbm, out_hbm, idx_v, rows_v, sem):
        wid = lax.axis_index("s") * nc + lax.axis_index("c")
        base = wid * b_per_w
        pltpu.sync_copy(idx_hbm.at[pl.ds(base, b_per_w)], idx_v)
        pltpu.async_copy(table_hbm.at[idx_v], rows_v, sem).wait()
        pltpu.sync_copy(rows_v, out_hbm.at[pl.ds(base, b_per_w)])

    return gather_kernel(codebook, indices)


def kernel(z, codebook):
    b, t, d = z.shape
    zf = z.reshape(-1, d)
    nn = zf.shape[0] // NT
    indices = _compute_indices(zf, codebook, 0, nn)
    quantized = _sc_gather(codebook, indices)
    return quantized.reshape(b, t, d), indices.reshape(b, t)
